# Initial kernel scaffold; baseline (speedup 1.0000x reference)
import jax
import jax.numpy as jnp
from jax.experimental import pallas as pl


def kernel(node_feats, edge_feats, params, edge_index):
    raise NotImplementedError("write your pallas kernel here")


# speedup vs baseline: 1.6110x; 1.6110x over previous
import functools

import jax
import jax.numpy as jnp
from jax import lax
from jax.experimental import pallas as pl
from jax.experimental.pallas import tpu as pltpu
from jax.experimental.pallas import tpu_sc as plsc

N = 10000
E = 320000
DN = 128
DE = 16
G = 128

NC = 2
NS = 16
NW = NC * NS
CH = 64
EPT = 10048
NCH = EPT // CH
E_PAD = NW * EPT
N_ACC = 10016
RPS = N_ACC // NS
ROWW = 144

_mesh = plsc.VectorSubcoreMesh(core_axis_name="c", subcore_axis_name="s")
_sc_params = pltpu.CompilerParams(use_tc_tiling_on_sc=False,
                                  needs_layout_passes=False)


def _lrelu(x):
    return jnp.maximum(x, 0.01 * x)


def _node_dense_body(nf_ref, wn_ref, bn_ref, we1n_ref, w2blk_ref, b2_ref,
                     hv_ref, u_ref, sn_ref):
    nf = nf_ref[...]
    hv = _lrelu(jnp.dot(nf, wn_ref[...], preferred_element_type=jnp.float32)
                + bn_ref[...][None, :])
    hv_ref[...] = hv
    u_ref[...] = jnp.dot(nf, we1n_ref[...], preferred_element_type=jnp.float32)
    sn_ref[...] = jnp.dot(hv, w2blk_ref[...],
                          preferred_element_type=jnp.float32) + b2_ref[...][None, :]


def _edge_v_body(ef_ref, we1e_ref, be1_ref, v_ref):
    v = jnp.dot(ef_ref[...], we1e_ref[...],
                preferred_element_type=jnp.float32) + be1_ref[...][None, :]
    eb = v.shape[0]
    v_ref[...] = jnp.transpose(v.reshape(eb // 16, 16, 3 * G), (0, 2, 1))


def _max_reduce_body(mpart_ref, m_ref):
    m = jnp.max(mpart_ref[...], axis=0)
    mt = jnp.transpose(m, (1, 0))
    m_ref[...] = jnp.concatenate(
        [mt, jnp.zeros((mt.shape[0], 13), jnp.float32)], axis=1)


def _take16(x, idx):
    return x.at[idx].get(mode="promise_in_bounds")


def _sc_pass1(u_hbm, v_hbm, sn_hbm, src_hbm, dst_hbm, w2_hbm,
              l_hbm, t_hbm, mpart_hbm,
              ubuf, vbuf, snbuf, mt, lst, srcb, dstb, w2b, sem, sem2):
    c = lax.axis_index("c")
    s = lax.axis_index("s")
    w = c * NS + s
    base0 = w * EPT
    grp00 = w * (EPT // 16)

    pltpu.sync_copy(w2_hbm, w2b)
    neg = jnp.full((16,), -1e30, jnp.float32)
    for i in range(3):
        def _init(j, _, i=i):
            mt[i, pl.ds(j * 16, 16)] = neg
            return 0
        lax.fori_loop(0, N // 16, _init, 0)

    lanes = lax.iota(jnp.int32, 16)

    def chunk_body(ch, _):
        base = base0 + ch * CH
        grp0 = grp00 + ch * (CH // 16)
        pltpu.sync_copy(src_hbm.at[pl.ds(base, CH)], srcb)
        pltpu.sync_copy(dst_hbm.at[pl.ds(base, CH)], dstb)
        pltpu.async_copy(u_hbm.at[srcb], ubuf, sem).wait()
        pltpu.async_copy(sn_hbm.at[dstb], snbuf, sem2).wait()
        pltpu.sync_copy(v_hbm.at[pl.ds(grp0, CH // 16)], vbuf)

        for g in range(CH // 16):
            ev = lanes + g * 16
            dv = dstb[pl.ds(g * 16, 16)]
            valid = (lanes + (base + g * 16)) < E
            for i in range(3):
                ihead = jnp.full((16,), i, jnp.int32)

                def f8_body(f8, acc, g=g, i=i, ev=ev):
                    fb = i * G + f8 * 16
                    w2vec = w2b[i, pl.ds(f8 * 16, 16)]
                    for k in range(16):
                        fcol = jnp.full((16,), fb + k, jnp.int32)
                        uvec = plsc.load_gather(ubuf, [ev, fcol])
                        gv = uvec + vbuf[g, fb + k]
                        t = jnp.maximum(gv, 0.01 * gv)
                        vbuf[g, fb + k] = t
                        acc = acc + t * w2vec[k]
                    return acc

                acc = lax.fori_loop(0, 8, f8_body, jnp.zeros((16,), jnp.float32))
                snv = plsc.load_gather(snbuf, [ev, ihead])
                z = snv + acc
                lg = jnp.maximum(z, 0.01 * z)
                lst[i, pl.ds(g * 16, 16)] = lg
                lg_eff = jnp.where(valid, lg, -1e30)
                sk, sv = plsc.sort_key_val(dv, lg_eff)
                for sh in (1, 2, 4, 8):
                    idx = jnp.maximum(lanes - sh, 0)
                    xk = _take16(sk, idx)
                    xv = _take16(sv, idx)
                    ok = (lanes >= sh) & (xk == sk)
                    sv = jnp.where(ok, jnp.maximum(sv, xv), sv)
                nxt = _take16(sk, jnp.minimum(lanes + 1, 15))
                last = (sk != nxt) | (lanes == 15)
                cur = plsc.load_gather(mt, [ihead, sk])
                plsc.store_scatter(mt, [ihead, sk], jnp.maximum(cur, sv),
                                   mask=last)

        pltpu.sync_copy(vbuf, t_hbm.at[pl.ds(grp0, CH // 16)])
        for i in range(3):
            pltpu.sync_copy(lst.at[i], l_hbm.at[i, pl.ds(base, CH)])
        return 0

    lax.fori_loop(0, NCH, chunk_body, 0)
    pltpu.sync_copy(mt, mpart_hbm.at[w])


def _sc_pass2(t_hbm, l_hbm, dst_hbm, m_hbm,
              pacc_hbm,
              acc, tbuf, rows, mbuf, lbuf, dstb, zbuf, sem, sem2):
    c = lax.axis_index("c")
    s = lax.axis_index("s")
    w = c * NS + s
    base0 = w * EPT
    grp00 = w * (EPT // 16)
    row0 = s * RPS

    zero16 = jnp.zeros((16,), jnp.float32)
    def _zb(j, _):
        r = j // (ROWW // 16)
        k = (j % (ROWW // 16)) * 16
        zbuf[r, pl.ds(k, 16)] = zero16
        return 0
    lax.fori_loop(0, CH * (ROWW // 16), _zb, 0)
    def _zr(e, _):
        rows[e, pl.ds(G, 16)] = zero16
        return 0
    lax.fori_loop(0, CH, _zr, 0)

    lanes = lax.iota(jnp.int32, 16)
    colex = jnp.full((16,), G, jnp.int32)

    for i in range(3):
        for j in range(RPS // CH):
            pltpu.sync_copy(zbuf, acc.at[pl.ds(row0 + j * CH, CH)])
        pltpu.sync_copy(zbuf.at[pl.ds(0, RPS - (RPS // CH) * CH)],
                        acc.at[pl.ds(row0 + (RPS // CH) * CH,
                                     RPS - (RPS // CH) * CH)])
        plsc.subcore_barrier()
        ihead = jnp.full((16,), i, jnp.int32)

        def chunk_body(ch, _, i=i, ihead=ihead):
            base = base0 + ch * CH
            grp0 = grp00 + ch * (CH // 16)
            pltpu.sync_copy(dst_hbm.at[pl.ds(base, CH)], dstb)
            pltpu.sync_copy(l_hbm.at[i, pl.ds(base, CH)], lbuf)
            pltpu.async_copy(m_hbm.at[dstb], mbuf, sem2).wait()
            pltpu.sync_copy(t_hbm.at[pl.ds(grp0, CH // 16), pl.ds(i * G, G)],
                            tbuf)
            for g in range(CH // 16):
                ev = lanes + g * 16
                mv = plsc.load_gather(mbuf, [ev, ihead])
                lv = lbuf[pl.ds(g * 16, 16)]
                ex = jnp.exp(lv - mv)
                mask = (lanes + (base + g * 16)) < E
                ex = jnp.where(mask, ex, 0.0)
                plsc.store_scatter(rows, [ev, colex], ex)

                def f8_body(f8, _, g=g, ev=ev, ex=ex):
                    for k in range(16):
                        f = f8 * 16 + k
                        tv = tbuf[g, f]
                        fcol = jnp.full((16,), f, jnp.int32)
                        plsc.store_scatter(rows, [ev, fcol], tv * ex)
                    return 0
                lax.fori_loop(0, 8, f8_body, 0)
            pltpu.sync_copy(rows, acc.at[dstb], add=True)
            return 0

        lax.fori_loop(0, NCH, chunk_body, 0)
        plsc.subcore_barrier()

        @pl.when(s < NS - 1)
        def _():
            pltpu.sync_copy(acc.at[pl.ds(row0, RPS)],
                            pacc_hbm.at[c, i, pl.ds(row0, RPS)])

        @pl.when(s == NS - 1)
        def _():
            pltpu.sync_copy(acc.at[pl.ds(row0, N - (NS - 1) * RPS)],
                            pacc_hbm.at[c, i, pl.ds(row0, N - (NS - 1) * RPS)])
        plsc.subcore_barrier()


def _epilogue_body(pacc_ref, hv_ref,
                   wet_ref, bet_ref, wmca_ref, bmca_ref, wmcn_ref, bmcn_ref,
                   wih_ref, bih_ref, whh_ref, bhh_ref,
                   out_ref):
    pacc = pacc_ref[...]
    hv = hv_ref[...]
    ctx = []
    for i in range(3):
        p = pacc[0, i] + pacc[1, i]
        pi = p[:, :G]
        si = p[:, G]
        re = jnp.where(si > 0, 1.0 / jnp.where(si > 0, si, 1.0), 0.0)
        a = pi * re[:, None]
        ci = jnp.dot(a, wet_ref[...][i], preferred_element_type=jnp.float32)
        ci = ci + jnp.where(si > 0, 1.0, 0.0)[:, None] * bet_ref[...][i][None, :]
        ctx.append(jnp.where(ci > 0, ci, jnp.exp(jnp.minimum(ci, 0.0)) - 1.0))
    context = jnp.dot(jnp.concatenate(ctx, axis=1), wmca_ref[...],
                      preferred_element_type=jnp.float32) + bmca_ref[...][None, :]
    hnode = jnp.dot(hv, wmcn_ref[...],
                    preferred_element_type=jnp.float32) + bmcn_ref[...][None, :]
    gi = jnp.dot(context, wih_ref[...],
                 preferred_element_type=jnp.float32) + bih_ref[...][None, :]
    gh = jnp.dot(hnode, whh_ref[...],
                 preferred_element_type=jnp.float32) + bhh_ref[...][None, :]
    i_r, i_z, i_n = gi[:, :G], gi[:, G:2 * G], gi[:, 2 * G:]
    h_r, h_z, h_n = gh[:, :G], gh[:, G:2 * G], gh[:, 2 * G:]
    r = jax.nn.sigmoid(i_r + h_r)
    z = jax.nn.sigmoid(i_z + h_z)
    cand = jnp.tanh(i_n + r * h_n)
    h_new = (1.0 - z) * cand + z * hnode
    out_ref[...] = jnp.maximum(h_new, 0.0)


def kernel(node_feats, edge_feats, params, edge_index):
    p = params
    f32 = jnp.float32

    wn_cat = jnp.concatenate([p['Wn%d' % i] for i in (1, 2, 3)], axis=1)
    bn_cat = jnp.concatenate([p['bn%d' % i] for i in (1, 2, 3)], axis=0)
    we1n_cat = jnp.concatenate([p['We1_%d' % i][:DN] for i in (1, 2, 3)], axis=1)
    we1e_cat = jnp.concatenate([p['We1_%d' % i][DN:] for i in (1, 2, 3)], axis=1)
    be1_cat = jnp.concatenate([p['be1_%d' % i] for i in (1, 2, 3)], axis=0)
    w2blk = jnp.zeros((3 * G, 16), f32)
    for i in (1, 2, 3):
        w2blk = w2blk.at[(i - 1) * G:i * G, i - 1].set(p['We2_%d' % i][:G, 0])
    b2 = jnp.zeros((16,), f32)
    for i in (1, 2, 3):
        b2 = b2.at[i - 1].set(p['be2_%d' % i][0])
    w2b = jnp.stack([p['We2_%d' % i][G:, 0] for i in (1, 2, 3)], axis=0)
    wet = jnp.stack([p['Wet%d' % i] for i in (1, 2, 3)], axis=0)
    bet = jnp.stack([p['bet%d' % i] for i in (1, 2, 3)], axis=0)

    src = jnp.pad(edge_index[0], (0, E_PAD - E)).astype(jnp.int32)
    dst = jnp.pad(edge_index[1], (0, E_PAD - E)).astype(jnp.int32)
    ef_pad = jnp.pad(edge_feats, ((0, E_PAD - E), (0, 0)))

    hv_cat, u_cat, sn3 = pl.pallas_call(
        _node_dense_body,
        out_shape=[jax.ShapeDtypeStruct((N, 3 * G), f32),
                   jax.ShapeDtypeStruct((N, 3 * G), f32),
                   jax.ShapeDtypeStruct((N, 16), f32)],
    )(node_feats, wn_cat, bn_cat, we1n_cat, w2blk, b2)

    EB = 2512
    v_cat = pl.pallas_call(
        _edge_v_body,
        grid=(E_PAD // EB,),
        in_specs=[pl.BlockSpec((EB, DE), lambda i: (i, 0)),
                  pl.BlockSpec((DE, 3 * G), lambda i: (0, 0)),
                  pl.BlockSpec((3 * G,), lambda i: (0,))],
        out_specs=pl.BlockSpec((EB // 16, 3 * G, 16), lambda i: (i, 0, 0)),
        out_shape=jax.ShapeDtypeStruct((E_PAD // 16, 3 * G, 16), f32),
        compiler_params=pltpu.CompilerParams(vmem_limit_bytes=100 * 2**20),
    )(ef_pad, we1e_cat, be1_cat)

    pass1 = pl.kernel(
        _sc_pass1,
        out_type=[jax.ShapeDtypeStruct((3, E_PAD), f32),
                  jax.ShapeDtypeStruct((E_PAD // 16, 3 * G, 16), f32),
                  jax.ShapeDtypeStruct((NW, 3, N), f32)],
        mesh=_mesh,
        scratch_types=[
            pltpu.VMEM((CH, 3 * G), f32),
            pltpu.VMEM((CH // 16, 3 * G, 16), f32),
            pltpu.VMEM((CH, 16), f32),
            pltpu.VMEM((3, N), f32),
            pltpu.VMEM((3, CH), f32),
            pltpu.VMEM((CH,), jnp.int32),
            pltpu.VMEM((CH,), jnp.int32),
            pltpu.VMEM((3, G), f32),
            pltpu.SemaphoreType.DMA,
            pltpu.SemaphoreType.DMA,
        ],
        compiler_params=_sc_params,
    )
    logits, he1t, mpart = pass1(u_cat, v_cat, sn3, src, dst, w2b)

    m3 = pl.pallas_call(
        _max_reduce_body,
        out_shape=jax.ShapeDtypeStruct((N, 16), f32),
    )(mpart)

    pass2 = pl.kernel(
        _sc_pass2,
        out_type=jax.ShapeDtypeStruct((NC, 3, N, ROWW), f32),
        mesh=_mesh,
        scratch_types=[
            pltpu.VMEM_SHARED((N_ACC, ROWW), f32),
            pltpu.VMEM((CH // 16, G, 16), f32),
            pltpu.VMEM((CH, ROWW), f32),
            pltpu.VMEM((CH, 16), f32),
            pltpu.VMEM((CH,), f32),
            pltpu.VMEM((CH,), jnp.int32),
            pltpu.VMEM((CH, ROWW), f32),
            pltpu.SemaphoreType.DMA,
            pltpu.SemaphoreType.DMA,
        ],
        compiler_params=_sc_params,
    )
    pacc = pass2(he1t, logits, dst, m3)

    NB = 2000
    out = pl.pallas_call(
        _epilogue_body,
        grid=(N // NB,),
        in_specs=[pl.BlockSpec((NC, 3, NB, ROWW), lambda k: (0, 0, k, 0)),
                  pl.BlockSpec((NB, 3 * G), lambda k: (k, 0)),
                  pl.BlockSpec((3, G, G), lambda k: (0, 0, 0)),
                  pl.BlockSpec((3, G), lambda k: (0, 0)),
                  pl.BlockSpec((3 * G, G), lambda k: (0, 0)),
                  pl.BlockSpec((G,), lambda k: (0,)),
                  pl.BlockSpec((3 * G, G), lambda k: (0, 0)),
                  pl.BlockSpec((G,), lambda k: (0,)),
                  pl.BlockSpec((G, 3 * G), lambda k: (0, 0)),
                  pl.BlockSpec((3 * G,), lambda k: (0,)),
                  pl.BlockSpec((G, 3 * G), lambda k: (0, 0)),
                  pl.BlockSpec((3 * G,), lambda k: (0,))],
        out_specs=pl.BlockSpec((NB, G), lambda k: (k, 0)),
        out_shape=jax.ShapeDtypeStruct((N, G), f32),
    )(pacc, hv_cat, wet, bet, p['Wmca'], p['bmca'], p['Wmcn'], p['bmcn'],
      p['W_ih'], p['b_ih'], p['W_hh'], p['b_hh'])
    return out
